# baseline (speedup 1.0000x reference)
import functools

import jax
import jax.numpy as jnp
from jax import lax
from jax.experimental import pallas as pl
from jax.experimental.pallas import tpu as pltpu
from jax.experimental.pallas import tpu_sc as plsc

_NUM_SC = 2
_NUM_SUBCORES = 16


def _pe_build_kernel(col_ref, row_ref, pe_ref):
    half = col_ref.shape[1]
    W = col_ref.shape[0]
    H = row_ref.shape[0]
    HW = H * W

    j = lax.broadcasted_iota(jnp.int32, (W, HW), 1)
    i = lax.broadcasted_iota(jnp.int32, (W, HW), 0)
    s_col = (lax.rem(j, W) == i).astype(jnp.float32)
    s_row = (lax.div(j, W) == i).astype(jnp.float32)
    col_t = col_ref[...].T
    row_t = row_ref[...].T
    pe_ref[:half] = lax.dot(
        col_t, s_col, precision=lax.Precision.HIGHEST,
        preferred_element_type=jnp.float32)
    pe_ref[half:] = lax.dot(
        row_t, s_row, precision=lax.Precision.HIGHEST,
        preferred_element_type=jnp.float32)


def _build_pe(col_table, row_table, C, HW):
    return pl.pallas_call(
        _pe_build_kernel,
        out_shape=jax.ShapeDtypeStruct((C, HW), jnp.float32),
    )(col_table, row_table)


def _sc_fanout(pe, B):
    C, HW = pe.shape
    NW = _NUM_SC * _NUM_SUBCORES
    c_per_w = C // NW

    mesh = plsc.VectorSubcoreMesh(
        core_axis_name="c", subcore_axis_name="s",
        num_cores=_NUM_SC, num_subcores=_NUM_SUBCORES)

    @functools.partial(
        pl.kernel,
        out_type=jax.ShapeDtypeStruct((B, C, HW), jnp.float32),
        mesh=mesh,
        scratch_types=[
            pltpu.VMEM((c_per_w, HW), jnp.float32),
            pltpu.SemaphoreType.DMA,
        ],
    )
    def fanout(pe_hbm, out_hbm, slice_v, sem):
        wid = lax.axis_index("s") * _NUM_SC + lax.axis_index("c")
        base = wid * c_per_w
        pltpu.sync_copy(pe_hbm.at[pl.ds(base, c_per_w)], slice_v)
        copies = [
            pltpu.make_async_copy(
                slice_v, out_hbm.at[b].at[pl.ds(base, c_per_w)], sem)
            for b in range(B)
        ]
        for cp in copies:
            cp.start()
        for cp in copies:
            cp.wait()

    return fanout(pe)


def kernel(image_feature, col_table, row_table):
    B, C, H, W = image_feature.shape
    pe = _build_pe(col_table, row_table, C, H * W)
    out = _sc_fanout(pe, B)
    return out


# speedup vs baseline: 3.1572x; 3.1572x over previous
import functools

import jax
import jax.numpy as jnp
from jax import lax
from jax.experimental import pallas as pl
from jax.experimental.pallas import tpu as pltpu
from jax.experimental.pallas import tpu_sc as plsc

_NUM_SC = 2
_NUM_SUBCORES = 16


def _pe_build_kernel(col_ref, row_ref, pe_ref):
    W, half = col_ref.shape
    H = row_ref.shape[0]
    col = col_ref[...]
    row = row_ref[...]
    pe_ref[:, :, :half] = jnp.broadcast_to(col[None, :, :], (H, W, half))
    pe_ref[:, :, half:] = jnp.broadcast_to(row[:, None, :], (H, W, half))


def _build_pe(col_table, row_table, H, W, C):
    return pl.pallas_call(
        _pe_build_kernel,
        out_shape=jax.ShapeDtypeStruct((H, W, C), jnp.float32),
    )(col_table, row_table)


def _sc_fanout(pe, B):
    HW, C = pe.shape
    N_RCHUNK = 8
    N_BGROUP = 4
    r_chunk = HW // N_RCHUNK
    b_group = B // N_BGROUP

    mesh = plsc.VectorSubcoreMesh(
        core_axis_name="c", subcore_axis_name="s",
        num_cores=_NUM_SC, num_subcores=_NUM_SUBCORES)

    @functools.partial(
        pl.kernel,
        out_type=jax.ShapeDtypeStruct((B, HW, C), jnp.float32),
        mesh=mesh,
        scratch_types=[
            pltpu.VMEM((r_chunk, C), jnp.float32),
            pltpu.SemaphoreType.DMA,
        ],
    )
    def fanout(pe_hbm, out_hbm, slice_v, sem):
        wid = lax.axis_index("s") * _NUM_SC + lax.axis_index("c")
        rchunk_id = lax.rem(wid, N_RCHUNK)
        bgroup_id = lax.div(wid, N_RCHUNK)
        base_r = rchunk_id * r_chunk
        base_b = bgroup_id * b_group
        pltpu.sync_copy(pe_hbm.at[pl.ds(base_r, r_chunk)], slice_v)
        copies = [
            pltpu.make_async_copy(
                slice_v, out_hbm.at[base_b + k].at[pl.ds(base_r, r_chunk)],
                sem)
            for k in range(b_group)
        ]
        for cp in copies:
            cp.start()
        for cp in copies:
            cp.wait()

    return fanout(pe)


def kernel(image_feature, col_table, row_table):
    B, C, H, W = image_feature.shape
    pe = _build_pe(col_table, row_table, H, W, C)
    out = _sc_fanout(pe.reshape(H * W, C), B)
    return out.reshape(B, H, W, C).transpose(0, 3, 1, 2)
